# baseline (speedup 1.0000x reference)
import dataclasses
import functools

import jax
import jax.numpy as jnp
from jax import lax
from jax.experimental import pallas as pl
from jax.experimental.pallas import tpu as pltpu
from jax.experimental.pallas import tpu_sc as plsc

_BETA = 0.66
_GAMMA = -0.1
_ZETA = 1.1

_N = 10000
_NPAD = 10240
_S = 128
_E = 320000
_NC = 2
_NS = 16
_NW = _NC * _NS
_B = 128
_E2 = 327680
_CHUNK = _E2 // _NW
_NB = _CHUNK // _B
_NB2 = _NB // 2


@functools.cache
def _mesh():
  return plsc.VectorSubcoreMesh(core_axis_name="c", subcore_axis_name="s",
                                num_cores=_NC, num_subcores=_NS)


def _sc_params():
  cp = pltpu.CompilerParams()
  if "needs_layout_passes" in pltpu.CompilerParams.__dataclass_fields__:
    cp = dataclasses.replace(cp, needs_layout_passes=False)
  return cp


def _f32(shape):
  return jax.ShapeDtypeStruct(shape, jnp.float32)


def _full16(x):
  return jnp.full((16,), x, jnp.int32)


def _k1_body(lnc_ref, mi_ref, m_ref, al_ref, ar_ref, bias_ref,
             arna_ref, a1_ref, a2_ref):
  r = jnp.concatenate([lnc_ref[...], mi_ref[...], m_ref[...]], axis=1)
  mean = jnp.mean(r, axis=1, keepdims=True)
  d = r - mean
  var = jnp.sum(d * d, axis=1, keepdims=True) / (_N - 1)
  rn = d * lax.rsqrt(var)
  arna_ref[...] = rn.T
  colsum = jnp.sum(rn, axis=0, keepdims=True)
  a1_ref[...] = colsum * al_ref[...]
  a2_ref[...] = colsum * ar_ref[...] + bias_ref[...]


def _k1(lnc, mi, m_, al_row, ar_row, bias11):
  return pl.pallas_call(
      _k1_body,
      out_shape=(_f32((_N, _S)), _f32((1, _N)), _f32((1, _N))),
  )(lnc, mi, m_, al_row, ar_row, bias11)


def _k2_body(src_hbm, dst_hbm, a1_hbm, a2_hbm, m_hbm, zp_hbm,
             src_v, dst_v, a1_v, a2_v, m_v, zp_v):
  wid = lax.axis_index("s") * _NC + lax.axis_index("c")
  base = wid * _CHUNK
  pltpu.sync_copy(src_hbm.at[pl.ds(base, _CHUNK)], src_v)
  pltpu.sync_copy(dst_hbm.at[pl.ds(base, _CHUNK)], dst_v)
  pltpu.sync_copy(a1_hbm, a1_v)
  pltpu.sync_copy(a2_hbm, a2_v)

  @pl.loop(0, _N // 16)
  def _(i):
    zp_v[pl.ds(i * 16, 16)] = jnp.zeros((16,), jnp.float32)

  lane = lax.iota(jnp.int32, 16)

  @pl.loop(0, _CHUNK // 16)
  def _(i):
    sv = src_v[pl.ds(i * 16, 16)]
    dv = dst_v[pl.ds(i * 16, 16)]
    logit = plsc.load_gather(a1_v, [sv]) + plsc.load_gather(a2_v, [dv])
    s = 1.0 / (1.0 + jnp.exp(logit * (-1.0 / _BETA)))
    sb = s * (_ZETA - _GAMMA) + _GAMMA
    mm = jnp.minimum(jnp.maximum(sb, 0.0), 1.0)
    mm = jnp.where(base + i * 16 + lane < _E, mm, 0.0)
    m_v[pl.ds(i * 16, 16)] = mm
    plsc.addupdate_scatter(zp_v, [dv], mm)

  pltpu.sync_copy(m_v, m_hbm.at[pl.ds(base, _CHUNK)])
  pltpu.sync_copy(zp_v, zp_hbm.at[pl.ds(wid * _N, _N)])


@functools.cache
def _k2():
  return pl.kernel(
      _k2_body,
      out_type=(_f32((_E2,)), _f32((_NW * _N,))),
      mesh=_mesh(),
      compiler_params=_sc_params(),
      scratch_types=[
          pltpu.VMEM((_CHUNK,), jnp.int32),
          pltpu.VMEM((_CHUNK,), jnp.int32),
          pltpu.VMEM((_N,), jnp.float32),
          pltpu.VMEM((_N,), jnp.float32),
          pltpu.VMEM((_CHUNK,), jnp.float32),
          pltpu.VMEM((_N,), jnp.float32),
      ],
  )


def _k2b_body(zp_ref, z_ref):
  z_ref[...] = jnp.sum(zp_ref[...], axis=0, keepdims=True)


def _k2b(zparts):
  return pl.pallas_call(_k2b_body, out_shape=_f32((1, _N)))(zparts)


def _round_prologue(rows_v, acc_sh, sid):
  @pl.loop(0, _B)
  def _(r):
    for q in range(8):
      rows_v[r, pl.ds(q * 16, 16)] = jnp.zeros((16,), jnp.float32)

  nrows = _NPAD // _NS
  rows0 = sid * nrows
  for k in range(nrows // _B):
    pltpu.sync_copy(rows_v, acc_sh.at[pl.ds(rows0 + k * _B, _B)])
  plsc.subcore_barrier()


def _scale_rows(ab_v, rows_v):
  @plsc.parallel_loop(0, _B, unroll=4)
  def _(r):
    av = plsc.load_gather(ab_v, [_full16(r)])
    for q in range(8):
      rows_v[r, pl.ds(q * 16, 16)] = rows_v[r, pl.ds(q * 16, 16)] * av


def _pipelined_round(fill, table_hbm, bufa, bufb, acc_sh):
  srcb_a, dstb_a, ab_a, rows_a, gsem_a, ssem_a = bufa
  srcb_b, dstb_b, ab_b, rows_b, gsem_b, ssem_b = bufb

  def gwait(srcb, rows, gsem):
    pltpu.make_async_copy(table_hbm.at[srcb], rows, gsem).wait()

  def cwait(rows, dstb, ssem):
    pltpu.make_async_copy(rows, acc_sh.at[dstb], ssem).wait()

  fill(0, srcb_a, dstb_a, ab_a)
  pltpu.async_copy(table_hbm.at[srcb_a], rows_a, gsem_a)

  @pl.loop(0, _NB2)
  def _(j2):
    @pl.when(j2 > 0)
    def _():
      cwait(rows_b, dstb_b, ssem_b)

    fill(2 * j2 + 1, srcb_b, dstb_b, ab_b)
    pltpu.async_copy(table_hbm.at[srcb_b], rows_b, gsem_b)

    gwait(srcb_a, rows_a, gsem_a)
    _scale_rows(ab_a, rows_a)
    pltpu.async_copy(rows_a, acc_sh.at[dstb_a], ssem_a, add=True)

    gwait(srcb_b, rows_b, gsem_b)
    _scale_rows(ab_b, rows_b)
    pltpu.async_copy(rows_b, acc_sh.at[dstb_b], ssem_b, add=True)

    @pl.when(j2 < _NB2 - 1)
    def _():
      cwait(rows_a, dstb_a, ssem_a)
      fill(2 * j2 + 2, srcb_a, dstb_a, ab_a)
      pltpu.async_copy(table_hbm.at[srcb_a], rows_a, gsem_a)

  cwait(rows_a, dstb_a, ssem_a)
  cwait(rows_b, dstb_b, ssem_b)


def _round_epilogue(acc_sh, part_hbm, cid, sid):
  plsc.subcore_barrier()
  nrows = _NPAD // _NS
  rows0 = sid * nrows
  pltpu.sync_copy(acc_sh.at[pl.ds(rows0, nrows)],
                  part_hbm.at[cid, pl.ds(rows0, nrows)])


def _buf_scratch():
  return [
      pltpu.VMEM((_B,), jnp.int32),
      pltpu.VMEM((_B,), jnp.int32),
      pltpu.VMEM((_B,), jnp.float32),
      pltpu.VMEM((_B, _S), jnp.float32),
      pltpu.SemaphoreType.DMA,
      pltpu.SemaphoreType.DMA,
  ]


def _round_scratch():
  return _buf_scratch() + _buf_scratch() + [
      pltpu.VMEM_SHARED((_NPAD, _S), jnp.float32),
      pltpu.SemaphoreType.DMA,
  ]


def _k3_body(src_hbm, dst_hbm, m_hbm, z_hbm, table_hbm, a_hbm, part_hbm,
             *scratch):
  bufa, bufb = scratch[0:6], scratch[6:12]
  acc_sh, fsem, mb_v, z_v = scratch[12:16]
  cid = lax.axis_index("c")
  sid = lax.axis_index("s")
  wid = sid * _NC + cid
  base = wid * _CHUNK
  pltpu.sync_copy(z_hbm, z_v)
  _round_prologue(bufa[3], acc_sh, sid)

  def fill(e, srcb, dstb, ab):
    b0 = base + e * _B
    d1 = pltpu.async_copy(src_hbm.at[pl.ds(b0, _B)], srcb, fsem)
    d2 = pltpu.async_copy(dst_hbm.at[pl.ds(b0, _B)], dstb, fsem)
    d3 = pltpu.async_copy(m_hbm.at[pl.ds(b0, _B)], mb_v, fsem)
    d1.wait(); d2.wait(); d3.wait()

    @pl.loop(0, _B, step=16)
    def _(kb):
      dv = dstb[pl.ds(kb, 16)]
      zs = plsc.load_gather(z_v, [dv])
      mm = mb_v[pl.ds(kb, 16)]
      ab[pl.ds(kb, 16)] = jnp.where(zs > 0.0, mm / zs, 0.0)

    pltpu.sync_copy(ab, a_hbm.at[pl.ds(b0, _B)])

  _pipelined_round(fill, table_hbm, bufa, bufb, acc_sh)
  _round_epilogue(acc_sh, part_hbm, cid, sid)


@functools.cache
def _k3():
  return pl.kernel(
      _k3_body,
      out_type=(_f32((_E2,)), _f32((_NC, _NPAD, _S))),
      mesh=_mesh(),
      compiler_params=_sc_params(),
      scratch_types=_round_scratch() + [
          pltpu.VMEM((_B,), jnp.float32),
          pltpu.VMEM((_N,), jnp.float32),
      ],
  )


def _k4_body(src_hbm, dst_hbm, a_hbm, table_hbm, part_hbm, *scratch):
  bufa, bufb = scratch[0:6], scratch[6:12]
  acc_sh, fsem = scratch[12:14]
  cid = lax.axis_index("c")
  sid = lax.axis_index("s")
  wid = sid * _NC + cid
  base = wid * _CHUNK
  _round_prologue(bufa[3], acc_sh, sid)

  def fill(e, srcb, dstb, ab):
    b0 = base + e * _B
    d1 = pltpu.async_copy(src_hbm.at[pl.ds(b0, _B)], srcb, fsem)
    d2 = pltpu.async_copy(dst_hbm.at[pl.ds(b0, _B)], dstb, fsem)
    d3 = pltpu.async_copy(a_hbm.at[pl.ds(b0, _B)], ab, fsem)
    d1.wait(); d2.wait(); d3.wait()

  _pipelined_round(fill, table_hbm, bufa, bufb, acc_sh)
  _round_epilogue(acc_sh, part_hbm, cid, sid)


@functools.cache
def _k4():
  return pl.kernel(
      _k4_body,
      out_type=_f32((_NC, _NPAD, _S)),
      mesh=_mesh(),
      compiler_params=_sc_params(),
      scratch_types=_round_scratch(),
  )


def _k3b_body(p_ref, out_ref):
  out_ref[...] = p_ref[0] + p_ref[1]


def _k3b(parts):
  return pl.pallas_call(_k3b_body, out_shape=_f32((_NPAD, _S)))(parts)


def _k5_body(ya_ref, yb_ref, q_ref, wmp_ref, mask_ref, bmp_ref,
             wph_ref, bph_ref, wpo_ref, bpo_ref, out_ref):
  y = ya_ref[...] + yb_ref[...] + q_ref[0] + q_ref[1]
  w1 = wmp_ref[...] * mask_ref[...]
  h1 = jnp.maximum(
      jnp.dot(w1, y, preferred_element_type=jnp.float32) + bmp_ref[...], 0.0)
  h2 = jnp.maximum(
      jnp.dot(wph_ref[...], h1, preferred_element_type=jnp.float32)
      + bph_ref[...], 0.0)
  h3 = jnp.dot(wpo_ref[...], h2,
               preferred_element_type=jnp.float32) + bpo_ref[...]
  row = lax.broadcasted_iota(jnp.int32, h3.shape, 0)
  lg = jnp.where(row < 2, h3, -jnp.inf)
  e = jnp.exp(lg - jnp.max(lg, axis=0, keepdims=True))
  sm = e / jnp.sum(e, axis=0, keepdims=True)
  out_ref[...] = sm[:2, :]


def _k5(ya, yb, q, wmp, mask, bmp, wph, bph, wpo, bpo):
  return pl.pallas_call(_k5_body, out_shape=_f32((2, _S)))(
      ya, yb, q, wmp, mask, bmp, wph, bph, wpo, bpo)


def kernel(lnc_data, mi_data, m_data, src, dst, attn_l, attn_r, bias_l0,
           DEmRNA_index, pathway_Mask, W_mp, b_mp, W_ph, b_ph, W_po, b_po):
  al_row = attn_l.reshape(1, _N)
  ar_row = attn_r.reshape(1, _N)
  bias11 = bias_l0.reshape(1, 1)
  a_rna, a1_row, a2_row = _k1(lnc_data, mi_data, m_data, al_row, ar_row,
                              bias11)
  a1 = a1_row.reshape(_N)
  a2 = a2_row.reshape(_N)

  src_p = jnp.pad(src, (0, _E2 - _E))
  dst_p = jnp.pad(dst, (0, _E2 - _E))
  m_e, zparts = _k2()(src_p, dst_p, a1, a2)
  z = _k2b(zparts.reshape(_NW, _N)).reshape(_N)

  a_e, part1 = _k3()(src_p, dst_p, m_e, z, a_rna)
  rna1 = _k3b(part1)
  part2 = _k4()(src_p, dst_p, a_e, rna1)

  de = DEmRNA_index.shape[0]
  start = DEmRNA_index[0]
  ya = lax.dynamic_slice(a_rna, (start, jnp.int32(0)), (de, _S))
  yb = lax.dynamic_slice(rna1, (start, jnp.int32(0)), (de, _S))
  q = lax.dynamic_slice(part2, (jnp.int32(0), start, jnp.int32(0)),
                        (_NC, de, _S))

  p, _de_w = W_mp.shape
  ph = W_ph.shape[0]
  p_pad, ph_pad = 512, 256
  wmp = jnp.pad(W_mp, ((0, p_pad - p), (0, 0)))
  mask = jnp.pad(pathway_Mask, ((0, p_pad - p), (0, 0)))
  bmp = jnp.pad(b_mp, (0, p_pad - p)).reshape(p_pad, 1)
  wph = jnp.pad(W_ph, ((0, ph_pad - ph), (0, p_pad - p)))
  bph = jnp.pad(b_ph, (0, ph_pad - ph)).reshape(ph_pad, 1)
  wpo = jnp.pad(W_po, ((0, 6), (0, ph_pad - ph)))
  bpo = jnp.pad(b_po, (0, 6)).reshape(8, 1)

  out = _k5(ya, yb, q, wmp, mask, bmp, wph, bph, wpo, bpo)
  return out.T


# speedup vs baseline: 2.7122x; 2.7122x over previous
import dataclasses
import functools

import jax
import jax.numpy as jnp
from jax import lax
from jax.experimental import pallas as pl
from jax.experimental.pallas import tpu as pltpu
from jax.experimental.pallas import tpu_sc as plsc

_BETA = 0.66
_GAMMA = -0.1
_ZETA = 1.1

_N = 10000
_NPAD = 10240
_S = 128
_E = 320000
_NC = 2
_NS = 16
_NW = _NC * _NS
_CHUNK = _E // _NW
_B = 80
_NB = _CHUNK // _B


@functools.cache
def _mesh():
  return plsc.VectorSubcoreMesh(core_axis_name="c", subcore_axis_name="s",
                                num_cores=_NC, num_subcores=_NS)


def _sc_params():
  cp = pltpu.CompilerParams()
  if "needs_layout_passes" in pltpu.CompilerParams.__dataclass_fields__:
    cp = dataclasses.replace(cp, needs_layout_passes=False)
  return cp


def _f32(shape):
  return jax.ShapeDtypeStruct(shape, jnp.float32)


def _full16(x):
  return jnp.full((16,), x, jnp.int32)


def _k1_body(lnc_ref, mi_ref, m_ref, al_ref, ar_ref, bias_ref,
             arna_ref, a1_ref, a2_ref):
  r = jnp.concatenate([lnc_ref[...], mi_ref[...], m_ref[...]], axis=1)
  mean = jnp.mean(r, axis=1, keepdims=True)
  d = r - mean
  var = jnp.sum(d * d, axis=1, keepdims=True) / (_N - 1)
  rn = d * lax.rsqrt(var)
  arna_ref[...] = rn.T
  colsum = jnp.sum(rn, axis=0, keepdims=True)
  a1_ref[...] = colsum * al_ref[...]
  a2_ref[...] = colsum * ar_ref[...] + bias_ref[...]


def _k1(lnc, mi, m_, al_row, ar_row, bias11):
  return pl.pallas_call(
      _k1_body,
      out_shape=(_f32((_N, _S)), _f32((1, _N)), _f32((1, _N))),
  )(lnc, mi, m_, al_row, ar_row, bias11)


def _k2_body(src_hbm, dst_hbm, a1_hbm, a2_hbm, m_hbm, zp_hbm,
             src_v, dst_v, a1_v, a2_v, m_v, zp_v):
  wid = lax.axis_index("s") * _NC + lax.axis_index("c")
  base = wid * _CHUNK
  pltpu.sync_copy(src_hbm.at[pl.ds(base, _CHUNK)], src_v)
  pltpu.sync_copy(dst_hbm.at[pl.ds(base, _CHUNK)], dst_v)
  pltpu.sync_copy(a1_hbm, a1_v)
  pltpu.sync_copy(a2_hbm, a2_v)

  @plsc.parallel_loop(0, _N // 16, unroll=4)
  def _(i):
    zp_v[pl.ds(i * 16, 16)] = jnp.zeros((16,), jnp.float32)

  @plsc.parallel_loop(0, _CHUNK // 16, unroll=4)
  def _(i):
    sv = src_v[pl.ds(i * 16, 16)]
    dv = dst_v[pl.ds(i * 16, 16)]
    logit = plsc.load_gather(a1_v, [sv]) + plsc.load_gather(a2_v, [dv])
    s = 1.0 / (1.0 + jnp.exp(logit * (-1.0 / _BETA)))
    sb = s * (_ZETA - _GAMMA) + _GAMMA
    mm = jnp.minimum(jnp.maximum(sb, 0.0), 1.0)
    m_v[pl.ds(i * 16, 16)] = mm
    plsc.addupdate_scatter(zp_v, [dv], mm)

  pltpu.sync_copy(m_v, m_hbm.at[pl.ds(base, _CHUNK)])
  pltpu.sync_copy(zp_v, zp_hbm.at[pl.ds(wid * _N, _N)])


@functools.cache
def _k2():
  return pl.kernel(
      _k2_body,
      out_type=(_f32((_E,)), _f32((_NW * _N,))),
      mesh=_mesh(),
      compiler_params=_sc_params(),
      scratch_types=[
          pltpu.VMEM((_CHUNK,), jnp.int32),
          pltpu.VMEM((_CHUNK,), jnp.int32),
          pltpu.VMEM((_N,), jnp.float32),
          pltpu.VMEM((_N,), jnp.float32),
          pltpu.VMEM((_CHUNK,), jnp.float32),
          pltpu.VMEM((_N,), jnp.float32),
      ],
  )


def _k2b_body(zp_ref, z_ref):
  z_ref[...] = jnp.sum(zp_ref[...], axis=0, keepdims=True)


def _k2b(zparts):
  return pl.pallas_call(_k2b_body, out_shape=_f32((1, _N)))(zparts)


def _round_prologue(rows_v, acc_sh, sid):
  @pl.loop(0, _B)
  def _(r):
    for q in range(8):
      rows_v[r, pl.ds(q * 16, 16)] = jnp.zeros((16,), jnp.float32)

  nrows = _NPAD // _NS
  rows0 = sid * nrows
  for k in range(nrows // _B):
    pltpu.sync_copy(rows_v, acc_sh.at[pl.ds(rows0 + k * _B, _B)])
  plsc.subcore_barrier()


def _scale_rows(ab_v, rows_v):
  @plsc.parallel_loop(0, _B, unroll=4)
  def _(r):
    av = plsc.load_gather(ab_v, [_full16(r)])
    for q in range(8):
      rows_v[r, pl.ds(q * 16, 16)] = rows_v[r, pl.ds(q * 16, 16)] * av


def _pipelined_round(fill, table_hbm, bufa, bufb, acc_sh):
  srcb_a, dstb_a, dsc_a, ab_a, rows_a, gsem_a, ssem_a = bufa
  srcb_b, dstb_b, dsc_b, ab_b, rows_b, gsem_b, ssem_b = bufb

  def gwait(srcb, rows, gsem):
    pltpu.make_async_copy(table_hbm.at[srcb], rows, gsem).wait()

  def scatter_start(rows, dstb, dsc, ssem):
    @plsc.parallel_loop(0, _B, step=16, unroll=2)
    def _(k):
      dsc[pl.ds(k, 16)] = dstb[pl.ds(k, 16)]

    pltpu.async_copy(rows, acc_sh.at[dsc], ssem, add=True)

  def cwait(rows, dsc, ssem):
    pltpu.make_async_copy(rows, acc_sh.at[dsc], ssem).wait()

  fill(0, srcb_a, dstb_a, ab_a)
  pltpu.async_copy(table_hbm.at[srcb_a], rows_a, gsem_a)

  @pl.loop(0, (_NB - 1) // 2)
  def _(j2):
    fill(2 * j2 + 1, srcb_b, dstb_b, ab_b)
    pltpu.async_copy(table_hbm.at[srcb_b], rows_b, gsem_b)

    gwait(srcb_a, rows_a, gsem_a)
    _scale_rows(ab_a, rows_a)
    scatter_start(rows_a, dstb_a, dsc_a, ssem_a)

    fill(2 * j2 + 2, srcb_a, dstb_a, ab_a)
    cwait(rows_a, dsc_a, ssem_a)
    pltpu.async_copy(table_hbm.at[srcb_a], rows_a, gsem_a)

    gwait(srcb_b, rows_b, gsem_b)
    _scale_rows(ab_b, rows_b)
    scatter_start(rows_b, dstb_b, dsc_b, ssem_b)
    cwait(rows_b, dsc_b, ssem_b)

  gwait(srcb_a, rows_a, gsem_a)
  _scale_rows(ab_a, rows_a)
  scatter_start(rows_a, dstb_a, dsc_a, ssem_a)
  cwait(rows_a, dsc_a, ssem_a)


def _round_epilogue(acc_sh, part_hbm, cid, sid):
  plsc.subcore_barrier()
  nrows = _NPAD // _NS
  rows0 = sid * nrows
  pltpu.sync_copy(acc_sh.at[pl.ds(rows0, nrows)],
                  part_hbm.at[cid, pl.ds(rows0, nrows)])


def _buf_scratch():
  return [
      pltpu.VMEM((_B,), jnp.int32),
      pltpu.VMEM((_B,), jnp.int32),
      pltpu.VMEM((_B,), jnp.int32),
      pltpu.VMEM((_B,), jnp.float32),
      pltpu.VMEM((_B, _S), jnp.float32),
      pltpu.SemaphoreType.DMA,
      pltpu.SemaphoreType.DMA,
  ]


def _round_scratch():
  return _buf_scratch() + _buf_scratch() + [
      pltpu.VMEM_SHARED((_NPAD, _S), jnp.float32),
      pltpu.SemaphoreType.DMA,
  ]


def _k3_body(src_hbm, dst_hbm, m_hbm, z_hbm, table_hbm, a_hbm, part_hbm,
             *scratch):
  bufa, bufb = scratch[0:7], scratch[7:14]
  acc_sh, fsem, mb_v, z_v = scratch[14:18]
  cid = lax.axis_index("c")
  sid = lax.axis_index("s")
  wid = sid * _NC + cid
  base = wid * _CHUNK
  pltpu.sync_copy(z_hbm, z_v)
  _round_prologue(bufa[4], acc_sh, sid)

  def fill(e, srcb, dstb, ab):
    b0 = base + e * _B
    d1 = pltpu.async_copy(src_hbm.at[pl.ds(b0, _B)], srcb, fsem)
    d2 = pltpu.async_copy(dst_hbm.at[pl.ds(b0, _B)], dstb, fsem)
    d3 = pltpu.async_copy(m_hbm.at[pl.ds(b0, _B)], mb_v, fsem)
    d1.wait(); d2.wait(); d3.wait()

    @pl.loop(0, _B, step=16)
    def _(kb):
      dv = dstb[pl.ds(kb, 16)]
      zs = plsc.load_gather(z_v, [dv])
      mm = mb_v[pl.ds(kb, 16)]
      ab[pl.ds(kb, 16)] = jnp.where(zs > 0.0, mm / zs, 0.0)

    pltpu.sync_copy(ab, a_hbm.at[pl.ds(b0, _B)])

  _pipelined_round(fill, table_hbm, bufa, bufb, acc_sh)
  _round_epilogue(acc_sh, part_hbm, cid, sid)


@functools.cache
def _k3():
  return pl.kernel(
      _k3_body,
      out_type=(_f32((_E,)), _f32((_NC, _NPAD, _S))),
      mesh=_mesh(),
      compiler_params=_sc_params(),
      scratch_types=_round_scratch() + [
          pltpu.VMEM((_B,), jnp.float32),
          pltpu.VMEM((_N,), jnp.float32),
      ],
  )


def _k4_body(src_hbm, dst_hbm, a_hbm, table_hbm, part_hbm, *scratch):
  bufa, bufb = scratch[0:7], scratch[7:14]
  acc_sh, fsem = scratch[14:16]
  cid = lax.axis_index("c")
  sid = lax.axis_index("s")
  wid = sid * _NC + cid
  base = wid * _CHUNK
  _round_prologue(bufa[4], acc_sh, sid)

  def fill(e, srcb, dstb, ab):
    b0 = base + e * _B
    d1 = pltpu.async_copy(src_hbm.at[pl.ds(b0, _B)], srcb, fsem)
    d2 = pltpu.async_copy(dst_hbm.at[pl.ds(b0, _B)], dstb, fsem)
    d3 = pltpu.async_copy(a_hbm.at[pl.ds(b0, _B)], ab, fsem)
    d1.wait(); d2.wait(); d3.wait()

  _pipelined_round(fill, table_hbm, bufa, bufb, acc_sh)
  _round_epilogue(acc_sh, part_hbm, cid, sid)


@functools.cache
def _k4():
  return pl.kernel(
      _k4_body,
      out_type=_f32((_NC, _NPAD, _S)),
      mesh=_mesh(),
      compiler_params=_sc_params(),
      scratch_types=_round_scratch(),
  )


def _k3b_body(p_ref, out_ref):
  out_ref[...] = p_ref[0] + p_ref[1]


def _k3b(parts):
  return pl.pallas_call(_k3b_body, out_shape=_f32((_NPAD, _S)))(parts)


def _k4b_body(wmp_ref, mask_ref, w1_ref):
  w1_ref[...] = wmp_ref[...] * mask_ref[...]


def _k4b(wmp, mask):
  return pl.pallas_call(_k4b_body, out_shape=_f32(wmp.shape))(wmp, mask)


def _k5_body(ya_ref, yb_ref, q_ref, w1_ref, bmp_ref,
             wph_ref, bph_ref, wpo_ref, bpo_ref, out_ref):
  y = ya_ref[...] + yb_ref[...] + q_ref[0] + q_ref[1]
  h1 = jnp.maximum(
      jnp.dot(w1_ref[...], y, preferred_element_type=jnp.float32)
      + bmp_ref[...], 0.0)
  h2 = jnp.maximum(
      jnp.dot(wph_ref[...], h1, preferred_element_type=jnp.float32)
      + bph_ref[...], 0.0)
  h3 = jnp.dot(wpo_ref[...], h2,
               preferred_element_type=jnp.float32) + bpo_ref[...]
  e = jnp.exp(h3 - jnp.max(h3, axis=0, keepdims=True))
  out_ref[...] = e / jnp.sum(e, axis=0, keepdims=True)


def _k5(ya, yb, q, w1, bmp, wph, bph, wpo, bpo):
  return pl.pallas_call(_k5_body, out_shape=_f32((2, _S)))(
      ya, yb, q, w1, bmp, wph, bph, wpo, bpo)


def kernel(lnc_data, mi_data, m_data, src, dst, attn_l, attn_r, bias_l0,
           DEmRNA_index, pathway_Mask, W_mp, b_mp, W_ph, b_ph, W_po, b_po):
  al_row = attn_l.reshape(1, _N)
  ar_row = attn_r.reshape(1, _N)
  bias11 = bias_l0.reshape(1, 1)
  a_rna, a1_row, a2_row = _k1(lnc_data, mi_data, m_data, al_row, ar_row,
                              bias11)
  a1 = a1_row.reshape(_N)
  a2 = a2_row.reshape(_N)

  w1 = _k4b(W_mp, pathway_Mask)
  m_e, zparts = _k2()(src, dst, a1, a2)
  z = _k2b(zparts.reshape(_NW, _N)).reshape(_N)

  a_e, part1 = _k3()(src, dst, m_e, z, a_rna)
  rna1 = _k3b(part1)
  part2 = _k4()(src, dst, a_e, rna1)

  de = DEmRNA_index.shape[0]
  start = DEmRNA_index[0]
  ya = lax.dynamic_slice(a_rna, (start, jnp.int32(0)), (de, _S))
  yb = lax.dynamic_slice(rna1, (start, jnp.int32(0)), (de, _S))
  q = lax.dynamic_slice(part2, (jnp.int32(0), start, jnp.int32(0)),
                        (_NC, de, _S))

  p = W_mp.shape[0]
  ph = W_ph.shape[0]
  out = _k5(ya, yb, q, w1, b_mp.reshape(p, 1),
            W_ph, b_ph.reshape(ph, 1), W_po, b_po.reshape(2, 1))
  return out.T
